# baseline (speedup 1.0000x reference)
import functools

import jax
import jax.numpy as jnp
from jax.experimental import pallas as pl
from jax.experimental.pallas import tpu as pltpu

N = 2048
D = 768
E = 8
K = 2
EPAD = 128
BT = 256


def _moe_body(x_ref, xbf_ref, wg_ref, bg_ref, we_ref, be_ref, out_ref,
              wfull_ref):
    e = pl.program_id(1)
    idxv = jax.lax.broadcasted_iota(jnp.int32, (BT, EPAD), 1)

    @pl.when(e == 0)
    def _router():
        logits = jnp.dot(x_ref[...], wg_ref[...],
                         preferred_element_type=jnp.float32) + bg_ref[...]
        m = jnp.max(logits, axis=1, keepdims=True)
        p = jnp.exp(logits - m)
        w = p / jnp.sum(p, axis=1, keepdims=True)
        m1 = jnp.max(w, axis=1, keepdims=True)
        i1 = jnp.min(jnp.where(w == m1, idxv, EPAD), axis=1, keepdims=True)
        wc = jnp.where(idxv == i1, -1.0, w)
        m2 = jnp.max(wc, axis=1, keepdims=True)
        i2 = jnp.min(jnp.where(wc == m2, idxv, EPAD), axis=1, keepdims=True)
        sel = (idxv == i1) | (idxv == i2)
        wsel = jnp.where(sel, w, 0.0)
        wfull = wsel / (jnp.sum(wsel, axis=1, keepdims=True) + 1e-10)
        wfull_ref[...] = wfull
        out_ref[...] = jnp.dot(wfull, be_ref[...],
                               preferred_element_type=jnp.float32)

    w_e = jnp.sum(jnp.where(idxv == e, wfull_ref[...], 0.0),
                  axis=1, keepdims=True)
    y = jnp.dot(xbf_ref[...], we_ref[0],
                preferred_element_type=jnp.float32)
    out_ref[...] += w_e * y


@jax.jit
def kernel(x, Wg, bg, We, be):
    wg_pad = jnp.zeros((D, EPAD), jnp.float32).at[:, :E].set(Wg)
    bg_pad = jnp.full((1, EPAD), -1e30, jnp.float32).at[0, :E].set(bg)
    be_pad = jnp.zeros((EPAD, D), jnp.float32).at[:E, :].set(be)
    x_bf = x.astype(jnp.bfloat16)
    we_bf = We.astype(jnp.bfloat16)

    grid = (N // BT, E)
    out = pl.pallas_call(
        _moe_body,
        grid=grid,
        in_specs=[
            pl.BlockSpec((BT, D), lambda t, e: (t, 0)),
            pl.BlockSpec((BT, D), lambda t, e: (t, 0)),
            pl.BlockSpec((D, EPAD), lambda t, e: (0, 0)),
            pl.BlockSpec((1, EPAD), lambda t, e: (0, 0)),
            pl.BlockSpec((1, D, D), lambda t, e: (e, 0, 0)),
            pl.BlockSpec((EPAD, D), lambda t, e: (0, 0)),
        ],
        out_specs=pl.BlockSpec((BT, D), lambda t, e: (t, 0)),
        out_shape=jax.ShapeDtypeStruct((N, D), jnp.float32),
        scratch_shapes=[pltpu.VMEM((BT, EPAD), jnp.float32)],
    )(x, x_bf, wg_pad, bg_pad, we_bf, be_pad)
    return out


# speedup vs baseline: 1.7477x; 1.7477x over previous
import jax
import jax.numpy as jnp
from jax.experimental import pallas as pl
from jax.experimental.pallas import tpu as pltpu

N = 2048
D = 768
E = 8
K = 2
EPAD = 128


def _moe_body(x_ref, xbf_ref, wg_ref, bg_ref, we_ref, be_ref, out_ref,
              wfull_ref):
    e = pl.program_id(0)
    idxv = jax.lax.broadcasted_iota(jnp.int32, (N, EPAD), 1)

    @pl.when(e == 0)
    def _router():
        logits = jnp.dot(x_ref[...], wg_ref[...],
                         preferred_element_type=jnp.float32) + bg_ref[...]
        m = jnp.max(logits, axis=1, keepdims=True)
        p = jnp.exp(logits - m)
        w = p / jnp.sum(p, axis=1, keepdims=True)
        m1 = jnp.max(w, axis=1, keepdims=True)
        i1 = jnp.min(jnp.where(w == m1, idxv, EPAD), axis=1, keepdims=True)
        wc = jnp.where(idxv == i1, -1.0, w)
        m2 = jnp.max(wc, axis=1, keepdims=True)
        i2 = jnp.min(jnp.where(wc == m2, idxv, EPAD), axis=1, keepdims=True)
        sel = (idxv == i1) | (idxv == i2)
        wsel = jnp.where(sel, w, 0.0)
        wfull = wsel / (jnp.sum(wsel, axis=1, keepdims=True) + 1e-10)
        wfull_ref[...] = wfull
        out_ref[...] = jnp.dot(wfull, be_ref[...],
                               preferred_element_type=jnp.float32)

    w_e = jnp.sum(jnp.where(idxv == e, wfull_ref[...], 0.0),
                  axis=1, keepdims=True)
    y = jnp.dot(xbf_ref[...], we_ref[0],
                preferred_element_type=jnp.float32)
    out_ref[...] += w_e * y


@jax.jit
def kernel(x, Wg, bg, We, be):
    wg_pad = jnp.zeros((D, EPAD), jnp.float32).at[:, :E].set(Wg)
    bg_pad = jnp.full((1, EPAD), -1e30, jnp.float32).at[0, :E].set(bg)
    be_pad = jnp.zeros((EPAD, D), jnp.float32).at[:E, :].set(be)
    x_bf = x.astype(jnp.bfloat16)
    we_bf = We.astype(jnp.bfloat16)

    out = pl.pallas_call(
        _moe_body,
        grid=(E,),
        in_specs=[
            pl.BlockSpec((N, D), lambda e: (0, 0)),
            pl.BlockSpec((N, D), lambda e: (0, 0)),
            pl.BlockSpec((D, EPAD), lambda e: (0, 0)),
            pl.BlockSpec((1, EPAD), lambda e: (0, 0)),
            pl.BlockSpec((1, D, D), lambda e: (e, 0, 0)),
            pl.BlockSpec((EPAD, D), lambda e: (0, 0)),
        ],
        out_specs=pl.BlockSpec((N, D), lambda e: (0, 0)),
        out_shape=jax.ShapeDtypeStruct((N, D), jnp.float32),
        scratch_shapes=[pltpu.VMEM((N, EPAD), jnp.float32)],
    )(x, x_bf, wg_pad, bg_pad, we_bf, be_pad)
    return out
